# Initial kernel scaffold; baseline (speedup 1.0000x reference)
import jax
import jax.numpy as jnp
from jax.experimental import pallas as pl


def kernel(x, edge_index, W, b):
    raise NotImplementedError("write your pallas kernel here")


# speedup vs baseline: 10.2479x; 10.2479x over previous
import functools

import jax
import jax.numpy as jnp
from jax import lax
from jax.experimental import pallas as pl
from jax.experimental.pallas import tpu as pltpu
from jax.experimental.pallas import tpu_sc as plsc

N = 10000
NP = 10240
E = 320000
D = 128
NC = 2
NS = 16
CW = 128
CH = 80
NB = 5
EPAD = NC * NS * CH * CW
PADNODE = N + 100
RPT = NP // NS
RB = 1024

_mesh = plsc.VectorSubcoreMesh(core_axis_name="c", subcore_axis_name="s")


@functools.partial(
    pl.kernel,
    out_type=jax.ShapeDtypeStruct((NC, NP, D), jnp.float32),
    mesh=_mesh,
    scratch_types=[
        pltpu.VMEM((CH, CW), jnp.int32),
        pltpu.VMEM((CW, D), jnp.float32),
        pltpu.VMEM((32, D), jnp.float32),
        pltpu.VMEM_SHARED((NP, D), jnp.float32),
    ],
)
def _deg_kernel(dst_hbm, out_hbm, dst_v, ones_v, zb_v, acc_sh):
    c = lax.axis_index("c")
    s = lax.axis_index("s")
    pltpu.sync_copy(dst_hbm.at[c, s], dst_v)

    one = jnp.ones((16,), jnp.float32)
    zero = jnp.zeros((16,), jnp.float32)

    def fill_ones(i, _):
        for q in range(8):
            ones_v[i, pl.ds(q * 16, 16)] = one
        return 0

    lax.fori_loop(0, CW, fill_ones, 0)

    def fill_zero(i, _):
        for q in range(8):
            zb_v[i, pl.ds(q * 16, 16)] = zero
        return 0

    lax.fori_loop(0, 32, fill_zero, 0)
    for t in range(RPT // 32):
        pltpu.sync_copy(zb_v, acc_sh.at[pl.ds(s * RPT + t * 32, 32)])
    plsc.subcore_barrier()

    def body(j, _):
        pltpu.sync_copy(ones_v, acc_sh.at[dst_v.at[j]], add=True)
        return 0

    lax.fori_loop(0, CH, body, 0)
    plsc.subcore_barrier()
    pltpu.sync_copy(acc_sh.at[pl.ds(s * RPT, RPT)],
                    out_hbm.at[c].at[pl.ds(s * RPT, RPT)])


@functools.partial(
    pl.kernel,
    out_type=jax.ShapeDtypeStruct((NC, NP, D), jnp.float32),
    mesh=_mesh,
    scratch_types=[
        pltpu.VMEM((CH, CW), jnp.int32),
        pltpu.VMEM((16, CW), jnp.int32),
        pltpu.VMEM((16, CW), jnp.int32),
        pltpu.VMEM((CW, D), jnp.float32),
        pltpu.VMEM((CW, D), jnp.float32),
        pltpu.VMEM((8, D), jnp.float32),
        pltpu.VMEM_SHARED((NP, D), jnp.float32),
        pltpu.SemaphoreType.DMA,
        pltpu.SemaphoreType.DMA,
        pltpu.SemaphoreType.DMA,
        pltpu.SemaphoreType.DMA,
    ],
)
def _agg_kernel(src_hbm, dst_hbm, u_hbm, out_hbm,
                src_v, dsl_a, dsl_b, rows_a, rows_b, zb_v, acc_sh,
                gsem_a, gsem_b, ssem_a, ssem_b):
    c = lax.axis_index("c")
    s = lax.axis_index("s")
    pltpu.sync_copy(src_hbm.at[c, s], src_v)

    zero = jnp.zeros((16,), jnp.float32)

    def fz(i, _):
        for q in range(8):
            zb_v[i, pl.ds(q * 16, 16)] = zero
        return 0

    lax.fori_loop(0, 8, fz, 0)
    for t in range(RPT // 8):
        pltpu.sync_copy(zb_v, acc_sh.at[pl.ds(s * RPT + t * 8, 8)])
    plsc.subcore_barrier()

    pltpu.async_copy(u_hbm.at[src_v.at[0]], rows_a, gsem_a)
    pltpu.async_copy(u_hbm.at[src_v.at[1]], rows_b, gsem_b)

    dslabs = (dsl_a, dsl_b)
    for b in range(NB):
        dsl = dslabs[b % 2]
        pltpu.sync_copy(dst_hbm.at[c, s].at[pl.ds(b * 16, 16)], dsl)

        def inner(t, _, b=b, dsl=dsl):
            j = b * 16 + 2 * t
            r = 2 * t
            pltpu.make_async_copy(u_hbm.at[src_v.at[j]], rows_a, gsem_a).wait()
            pltpu.async_copy(rows_a, acc_sh.at[dsl.at[r]], ssem_a, add=True)
            pltpu.make_async_copy(u_hbm.at[src_v.at[j + 1]], rows_b,
                                  gsem_b).wait()
            pltpu.async_copy(rows_b, acc_sh.at[dsl.at[r + 1]], ssem_b,
                             add=True)
            pltpu.make_async_copy(rows_a, acc_sh.at[dsl.at[r]], ssem_a).wait()

            @pl.when(j + 2 < CH)
            def _():
                pltpu.async_copy(u_hbm.at[src_v.at[j + 2]], rows_a, gsem_a)

            pltpu.make_async_copy(rows_b, acc_sh.at[dsl.at[r + 1]],
                                  ssem_b).wait()

            @pl.when(j + 3 < CH)
            def _():
                pltpu.async_copy(u_hbm.at[src_v.at[j + 3]], rows_b, gsem_b)

            return 0

        lax.fori_loop(0, 8, inner, 0)

    plsc.subcore_barrier()
    pltpu.sync_copy(acc_sh.at[pl.ds(s * RPT, RPT)],
                    out_hbm.at[c].at[pl.ds(s * RPT, RPT)])


def _deg_col(degp_ref):
    return degp_ref[0, :, 0:1] + degp_ref[1, :, 0:1] + 1.0


def _scale_in_body(degp_ref, x_ref, u_ref):
    u_ref[...] = x_ref[...] * lax.rsqrt(_deg_col(degp_ref))


def _mid_body(degp_ref, u_ref, p_ref, v_ref):
    v_ref[...] = (u_ref[...] + p_ref[0] + p_ref[1]) / _deg_col(degp_ref)


def _out_body(degp_ref, v_ref, q_ref, w_ref, b_ref, o_ref):
    h2 = (v_ref[...] + q_ref[0] + q_ref[1]) * lax.rsqrt(_deg_col(degp_ref))
    o_ref[...] = lax.dot_general(
        h2, w_ref[...], (((1,), (1,)), ((), ())),
        preferred_element_type=jnp.float32) + b_ref[...]


_degp_spec = pl.BlockSpec((NC, RB, D), lambda i: (0, i, 0))
_row_spec = pl.BlockSpec((RB, D), lambda i: (i, 0))
_pair_spec = pl.BlockSpec((NC, RB, D), lambda i: (0, i, 0))

_tc_scale_in = pl.pallas_call(
    _scale_in_body,
    grid=(NP // RB,),
    in_specs=[_degp_spec, _row_spec],
    out_specs=_row_spec,
    out_shape=jax.ShapeDtypeStruct((NP, D), jnp.float32),
)

_tc_mid = pl.pallas_call(
    _mid_body,
    grid=(NP // RB,),
    in_specs=[_degp_spec, _row_spec, _pair_spec],
    out_specs=_row_spec,
    out_shape=jax.ShapeDtypeStruct((NP, D), jnp.float32),
)

_tc_out = pl.pallas_call(
    _out_body,
    grid=(NP // RB,),
    in_specs=[
        _degp_spec, _row_spec, _pair_spec,
        pl.BlockSpec((D, D), lambda i: (0, 0)),
        pl.BlockSpec((1, D), lambda i: (0, 0)),
    ],
    out_specs=_row_spec,
    out_shape=jax.ShapeDtypeStruct((NP, D), jnp.float32),
)


@jax.jit
def kernel(x, edge_index, W, b):
    pad = jnp.full((EPAD - E,), PADNODE, dtype=edge_index.dtype)
    src3 = jnp.concatenate([edge_index[0], pad]).reshape(NC, NS, CH, CW)
    dst3 = jnp.concatenate([edge_index[1], pad]).reshape(NC, NS, CH, CW)
    x_pad = jnp.pad(x, ((0, NP - N), (0, 0)))

    degp = _deg_kernel(dst3)
    u = _tc_scale_in(degp, x_pad)
    p = _agg_kernel(src3, dst3, u)
    v = _tc_mid(degp, u, p)
    q = _agg_kernel(src3, dst3, v)
    out = _tc_out(degp, v, q, W, b.reshape(1, D))
    return out[:N]


# speedup vs baseline: 10.6267x; 1.0370x over previous
import functools

import jax
import jax.numpy as jnp
from jax import lax
from jax.experimental import pallas as pl
from jax.experimental.pallas import tpu as pltpu
from jax.experimental.pallas import tpu_sc as plsc

N = 10000
NP = 10240
E = 320000
D = 128
NC = 2
NS = 16
CW = 128
CH = 80
NB = 5
EPAD = NC * NS * CH * CW
PADNODE = N + 100
RPT = NP // NS
RB = 1024

_mesh = plsc.VectorSubcoreMesh(core_axis_name="c", subcore_axis_name="s")


@functools.partial(
    pl.kernel,
    out_type=jax.ShapeDtypeStruct((NC, NP, D), jnp.float32),
    mesh=_mesh,
    scratch_types=[
        pltpu.VMEM((CH, CW), jnp.int32),
        pltpu.VMEM((CW, D), jnp.float32),
        pltpu.VMEM((32, D), jnp.float32),
        pltpu.VMEM_SHARED((NP, D), jnp.float32),
    ],
)
def _deg_kernel(dst_hbm, out_hbm, dst_v, ones_v, zb_v, acc_sh):
    c = lax.axis_index("c")
    s = lax.axis_index("s")
    pltpu.sync_copy(dst_hbm.at[c, s], dst_v)

    one = jnp.ones((16,), jnp.float32)
    zero = jnp.zeros((16,), jnp.float32)

    def fill_ones(i, _):
        for q in range(8):
            ones_v[i, pl.ds(q * 16, 16)] = one
        return 0

    lax.fori_loop(0, CW, fill_ones, 0)

    def fill_zero(i, _):
        for q in range(8):
            zb_v[i, pl.ds(q * 16, 16)] = zero
        return 0

    lax.fori_loop(0, 32, fill_zero, 0)
    for t in range(RPT // 32):
        pltpu.sync_copy(zb_v, acc_sh.at[pl.ds(s * RPT + t * 32, 32)])
    plsc.subcore_barrier()

    def body(j, _):
        pltpu.sync_copy(ones_v, acc_sh.at[dst_v.at[j]], add=True)
        return 0

    lax.fori_loop(0, CH, body, 0)
    plsc.subcore_barrier()
    pltpu.sync_copy(acc_sh.at[pl.ds(s * RPT, RPT)],
                    out_hbm.at[c].at[pl.ds(s * RPT, RPT)])


@functools.partial(
    pl.kernel,
    out_type=jax.ShapeDtypeStruct((NC, NP, D), jnp.float32),
    mesh=_mesh,
    scratch_types=[
        pltpu.VMEM((CH, CW), jnp.int32),
        pltpu.VMEM((16, CW), jnp.int32),
        pltpu.VMEM((16, CW), jnp.int32),
        pltpu.VMEM((CW, D), jnp.float32),
        pltpu.VMEM((CW, D), jnp.float32),
        pltpu.VMEM((8, D), jnp.float32),
        pltpu.VMEM_SHARED((NP, D), jnp.float32),
        pltpu.SemaphoreType.DMA,
        pltpu.SemaphoreType.DMA,
    ],
)
def _agg_kernel(src_hbm, dst_hbm, u_hbm, out_hbm,
                src_v, dsl_a, dsl_b, rows_a, rows_b, zb_v, acc_sh,
                gsem_a, gsem_b):
    c = lax.axis_index("c")
    s = lax.axis_index("s")
    pltpu.sync_copy(src_hbm.at[c, s], src_v)

    zero = jnp.zeros((16,), jnp.float32)

    def fz(i, _):
        for q in range(8):
            zb_v[i, pl.ds(q * 16, 16)] = zero
        return 0

    lax.fori_loop(0, 8, fz, 0)
    for t in range(RPT // 8):
        pltpu.sync_copy(zb_v, acc_sh.at[pl.ds(s * RPT + t * 8, 8)])
    plsc.subcore_barrier()

    pltpu.async_copy(u_hbm.at[src_v.at[0]], rows_a, gsem_a)
    pltpu.async_copy(u_hbm.at[src_v.at[1]], rows_b, gsem_b)

    dslabs = (dsl_a, dsl_b)
    for b in range(NB):
        dsl = dslabs[b % 2]
        pltpu.sync_copy(dst_hbm.at[c, s].at[pl.ds(b * 16, 16)], dsl)

        def inner(t, _, b=b, dsl=dsl):
            j = b * 16 + 2 * t
            r = 2 * t
            pltpu.make_async_copy(u_hbm.at[src_v.at[j]], rows_a, gsem_a).wait()
            pltpu.sync_copy(rows_a, acc_sh.at[dsl.at[r]], add=True)

            @pl.when(j + 2 < CH)
            def _():
                pltpu.async_copy(u_hbm.at[src_v.at[j + 2]], rows_a, gsem_a)

            pltpu.make_async_copy(u_hbm.at[src_v.at[j + 1]], rows_b,
                                  gsem_b).wait()
            pltpu.sync_copy(rows_b, acc_sh.at[dsl.at[r + 1]], add=True)

            @pl.when(j + 3 < CH)
            def _():
                pltpu.async_copy(u_hbm.at[src_v.at[j + 3]], rows_b, gsem_b)

            return 0

        lax.fori_loop(0, 8, inner, 0)

    plsc.subcore_barrier()
    pltpu.sync_copy(acc_sh.at[pl.ds(s * RPT, RPT)],
                    out_hbm.at[c].at[pl.ds(s * RPT, RPT)])


def _deg_col(degp_ref):
    return degp_ref[0, :, 0:1] + degp_ref[1, :, 0:1] + 1.0


def _scale_in_body(degp_ref, x_ref, u_ref):
    u_ref[...] = x_ref[...] * lax.rsqrt(_deg_col(degp_ref))


def _mid_body(degp_ref, u_ref, p_ref, v_ref):
    v_ref[...] = (u_ref[...] + p_ref[0] + p_ref[1]) / _deg_col(degp_ref)


def _out_body(degp_ref, v_ref, q_ref, w_ref, b_ref, o_ref):
    h2 = (v_ref[...] + q_ref[0] + q_ref[1]) * lax.rsqrt(_deg_col(degp_ref))
    o_ref[...] = lax.dot_general(
        h2, w_ref[...], (((1,), (1,)), ((), ())),
        preferred_element_type=jnp.float32) + b_ref[...]


_degp_spec = pl.BlockSpec((NC, RB, D), lambda i: (0, i, 0))
_row_spec = pl.BlockSpec((RB, D), lambda i: (i, 0))
_pair_spec = pl.BlockSpec((NC, RB, D), lambda i: (0, i, 0))

_tc_scale_in = pl.pallas_call(
    _scale_in_body,
    grid=(NP // RB,),
    in_specs=[_degp_spec, _row_spec],
    out_specs=_row_spec,
    out_shape=jax.ShapeDtypeStruct((NP, D), jnp.float32),
)

_tc_mid = pl.pallas_call(
    _mid_body,
    grid=(NP // RB,),
    in_specs=[_degp_spec, _row_spec, _pair_spec],
    out_specs=_row_spec,
    out_shape=jax.ShapeDtypeStruct((NP, D), jnp.float32),
)

_tc_out = pl.pallas_call(
    _out_body,
    grid=(NP // RB,),
    in_specs=[
        _degp_spec, _row_spec, _pair_spec,
        pl.BlockSpec((D, D), lambda i: (0, 0)),
        pl.BlockSpec((1, D), lambda i: (0, 0)),
    ],
    out_specs=_row_spec,
    out_shape=jax.ShapeDtypeStruct((NP, D), jnp.float32),
)


@jax.jit
def kernel(x, edge_index, W, b):
    pad = jnp.full((EPAD - E,), PADNODE, dtype=edge_index.dtype)
    src3 = jnp.concatenate([edge_index[0], pad]).reshape(NC, NS, CH, CW)
    dst3 = jnp.concatenate([edge_index[1], pad]).reshape(NC, NS, CH, CW)
    x_pad = jnp.pad(x, ((0, NP - N), (0, 0)))

    degp = _deg_kernel(dst3)
    u = _tc_scale_in(degp, x_pad)
    p = _agg_kernel(src3, dst3, u)
    v = _tc_mid(degp, u, p)
    q = _agg_kernel(src3, dst3, v)
    out = _tc_out(degp, v, q, W, b.reshape(1, D))
    return out[:N]


# speedup vs baseline: 27.7582x; 2.6121x over previous
import functools

import jax
import jax.numpy as jnp
from jax import lax
from jax.experimental import pallas as pl
from jax.experimental.pallas import tpu as pltpu
from jax.experimental.pallas import tpu_sc as plsc

N = 10000
NP = 10240
E = 320000
D = 128
NC = 2
NS = 16
CW = 128
CH = 80
NB = 5
EPAD = NC * NS * CH * CW
PADNODE = N + 100
RPT = NP // NS
RB = 1024

_mesh = plsc.VectorSubcoreMesh(core_axis_name="c", subcore_axis_name="s")


@functools.partial(
    pl.kernel,
    out_type=jax.ShapeDtypeStruct((NC, NP, D), jnp.float32),
    mesh=_mesh,
    scratch_types=[
        pltpu.VMEM((CH, CW), jnp.int32),
        pltpu.VMEM((CW, D), jnp.float32),
        pltpu.VMEM((32, D), jnp.float32),
        pltpu.VMEM_SHARED((NP, D), jnp.float32),
    ],
)
def _deg_kernel(dst_hbm, out_hbm, dst_v, ones_v, zb_v, acc_sh):
    c = lax.axis_index("c")
    s = lax.axis_index("s")
    pltpu.sync_copy(dst_hbm.at[c, s], dst_v)

    one = jnp.ones((16,), jnp.float32)
    zero = jnp.zeros((16,), jnp.float32)

    def fill_ones(i, _):
        for q in range(8):
            ones_v[i, pl.ds(q * 16, 16)] = one
        return 0

    lax.fori_loop(0, CW, fill_ones, 0)

    def fill_zero(i, _):
        for q in range(8):
            zb_v[i, pl.ds(q * 16, 16)] = zero
        return 0

    lax.fori_loop(0, 32, fill_zero, 0)
    for t in range(RPT // 32):
        pltpu.sync_copy(zb_v, acc_sh.at[pl.ds(s * RPT + t * 32, 32)])
    plsc.subcore_barrier()

    def body(j, _):
        pltpu.sync_copy(ones_v, acc_sh.at[dst_v.at[j]], add=True)
        return 0

    lax.fori_loop(0, CH, body, 0)
    plsc.subcore_barrier()
    pltpu.sync_copy(acc_sh.at[pl.ds(s * RPT, RPT)],
                    out_hbm.at[c].at[pl.ds(s * RPT, RPT)])


@functools.partial(
    pl.kernel,
    out_type=jax.ShapeDtypeStruct((NC, NP, D), jnp.float32),
    mesh=_mesh,
    scratch_types=[
        pltpu.VMEM((CH, CW), jnp.int32),
        pltpu.VMEM((16, CW), jnp.int32),
        pltpu.VMEM((16, CW), jnp.int32),
        pltpu.VMEM((CW, D), jnp.float32),
        pltpu.VMEM((CW, D), jnp.float32),
        pltpu.VMEM((8, D), jnp.float32),
        pltpu.VMEM_SHARED((NP, D), jnp.float32),
        pltpu.SemaphoreType.DMA,
        pltpu.SemaphoreType.DMA,
    ],
)
def _agg_kernel(src_hbm, dst_hbm, u_hbm, out_hbm,
                src_v, dsl_a, dsl_b, rows_a, rows_b, zb_v, acc_sh,
                gsem_a, gsem_b):
    c = lax.axis_index("c")
    s = lax.axis_index("s")
    pltpu.sync_copy(src_hbm.at[c, s], src_v)

    zero = jnp.zeros((16,), jnp.float32)

    def fz(i, _):
        for q in range(8):
            zb_v[i, pl.ds(q * 16, 16)] = zero
        return 0

    lax.fori_loop(0, 8, fz, 0)
    for t in range(RPT // 8):
        pltpu.sync_copy(zb_v, acc_sh.at[pl.ds(s * RPT + t * 8, 8)])
    plsc.subcore_barrier()

    pltpu.async_copy(u_hbm.at[src_v.at[0]], rows_a, gsem_a)
    pltpu.async_copy(u_hbm.at[src_v.at[1]], rows_b, gsem_b)

    dslabs = (dsl_a, dsl_b)
    for b in range(NB):
        dsl = dslabs[b % 2]
        pltpu.sync_copy(dst_hbm.at[c, s].at[pl.ds(b * 16, 16)], dsl)

        def inner(t, _, b=b, dsl=dsl):
            j = b * 16 + 2 * t
            r = 2 * t
            pltpu.make_async_copy(u_hbm.at[src_v.at[j]], rows_a, gsem_a).wait()
            pltpu.sync_copy(rows_a, acc_sh.at[dsl.at[r]], add=True)

            @pl.when(j + 2 < CH)
            def _():
                pltpu.async_copy(u_hbm.at[src_v.at[j + 2]], rows_a, gsem_a)

            pltpu.make_async_copy(u_hbm.at[src_v.at[j + 1]], rows_b,
                                  gsem_b).wait()
            pltpu.sync_copy(rows_b, acc_sh.at[dsl.at[r + 1]], add=True)

            @pl.when(j + 3 < CH)
            def _():
                pltpu.async_copy(u_hbm.at[src_v.at[j + 3]], rows_b, gsem_b)

            return 0

        lax.fori_loop(0, 8, inner, 0)

    plsc.subcore_barrier()
    pltpu.sync_copy(acc_sh.at[pl.ds(s * RPT, RPT)],
                    out_hbm.at[c].at[pl.ds(s * RPT, RPT)])


def _deg_col(degp_ref):
    return degp_ref[0, :, 0:1] + degp_ref[1, :, 0:1] + 1.0


def _scale_in_body(degp_ref, x_ref, u_ref):
    u_ref[...] = x_ref[...] * lax.rsqrt(_deg_col(degp_ref))


def _mid_body(degp_ref, u_ref, p_ref, v_ref):
    v_ref[...] = (u_ref[...] + p_ref[0] + p_ref[1]) / _deg_col(degp_ref)


def _out_body(degp_ref, v_ref, q_ref, w_ref, b_ref, o_ref):
    h2 = (v_ref[...] + q_ref[0] + q_ref[1]) * lax.rsqrt(_deg_col(degp_ref))
    o_ref[...] = lax.dot_general(
        h2, w_ref[...], (((1,), (1,)), ((), ())),
        preferred_element_type=jnp.float32) + b_ref[...]


_degp_spec = pl.BlockSpec((NC, RB, D), lambda i: (0, i, 0))
_row_spec = pl.BlockSpec((RB, D), lambda i: (i, 0))
_pair_spec = pl.BlockSpec((NC, RB, D), lambda i: (0, i, 0))

_tc_scale_in = pl.pallas_call(
    _scale_in_body,
    grid=(NP // RB,),
    in_specs=[_degp_spec, _row_spec],
    out_specs=_row_spec,
    out_shape=jax.ShapeDtypeStruct((NP, D), jnp.float32),
)

_tc_mid = pl.pallas_call(
    _mid_body,
    grid=(NP // RB,),
    in_specs=[_degp_spec, _row_spec, _pair_spec],
    out_specs=_row_spec,
    out_shape=jax.ShapeDtypeStruct((NP, D), jnp.float32),
)

_tc_out = pl.pallas_call(
    _out_body,
    grid=(NP // RB,),
    in_specs=[
        _degp_spec, _row_spec, _pair_spec,
        pl.BlockSpec((D, D), lambda i: (0, 0)),
        pl.BlockSpec((1, D), lambda i: (0, 0)),
    ],
    out_specs=_row_spec,
    out_shape=jax.ShapeDtypeStruct((NP, D), jnp.float32),
)


@jax.jit
def kernel(x, edge_index, W, b):
    ppw = CH * CW - E // (NC * NS)
    pad = jnp.broadcast_to(
        N + jnp.arange(ppw, dtype=edge_index.dtype), (NC, NS, ppw))
    src3 = jnp.concatenate(
        [edge_index[0].reshape(NC, NS, E // (NC * NS)), pad],
        axis=2).reshape(NC, NS, CH, CW)
    dst3 = jnp.concatenate(
        [edge_index[1].reshape(NC, NS, E // (NC * NS)), pad],
        axis=2).reshape(NC, NS, CH, CW)
    x_pad = jnp.pad(x, ((0, NP - N), (0, 0)))

    degp = _deg_kernel(dst3)
    u = _tc_scale_in(degp, x_pad)
    p = _agg_kernel(src3, dst3, u)
    v = _tc_mid(degp, u, p)
    q = _agg_kernel(src3, dst3, v)
    out = _tc_out(degp, v, q, W, b.reshape(1, D))
    return out[:N]


# speedup vs baseline: 28.9706x; 1.0437x over previous
import functools

import jax
import jax.numpy as jnp
from jax import lax
from jax.experimental import pallas as pl
from jax.experimental.pallas import tpu as pltpu
from jax.experimental.pallas import tpu_sc as plsc

N = 10000
NP = 10240
E = 320000
D = 128
NC = 2
NS = 16
CW = 128
CH = 80
NB = 5
EPAD = NC * NS * CH * CW
PADNODE = N + 100
RPT = NP // NS
RB = 1024

_mesh = plsc.VectorSubcoreMesh(core_axis_name="c", subcore_axis_name="s")


@functools.partial(
    pl.kernel,
    out_type=jax.ShapeDtypeStruct((NC, NP, D), jnp.float32),
    mesh=_mesh,
    scratch_types=[
        pltpu.VMEM((CH, CW), jnp.int32),
        pltpu.VMEM((CW, D), jnp.float32),
        pltpu.VMEM((CW, D), jnp.float32),
        pltpu.VMEM_SHARED((NP, D), jnp.float32),
    ],
)
def _deg_kernel(dst_hbm, out_hbm, dst_v, ones_v, zb_v, acc_sh):
    c = lax.axis_index("c")
    s = lax.axis_index("s")
    pltpu.sync_copy(dst_hbm.at[c, s], dst_v)

    one = jnp.ones((16,), jnp.float32)
    zero = jnp.zeros((16,), jnp.float32)

    def fill_ones(i, _):
        for q in range(8):
            ones_v[i, pl.ds(q * 16, 16)] = one
            zb_v[i, pl.ds(q * 16, 16)] = zero
        return 0

    lax.fori_loop(0, CW, fill_ones, 0)
    for t in range(RPT // CW):
        pltpu.sync_copy(zb_v, acc_sh.at[pl.ds(s * RPT + t * CW, CW)])
    plsc.subcore_barrier()

    def body(j, _):
        pltpu.sync_copy(ones_v, acc_sh.at[dst_v.at[j]], add=True)
        return 0

    lax.fori_loop(0, CH, body, 0)
    plsc.subcore_barrier()
    pltpu.sync_copy(acc_sh.at[pl.ds(s * RPT, RPT)],
                    out_hbm.at[c].at[pl.ds(s * RPT, RPT)])


@functools.partial(
    pl.kernel,
    out_type=jax.ShapeDtypeStruct((NC, NP, D), jnp.float32),
    mesh=_mesh,
    scratch_types=[
        pltpu.VMEM((CH, CW), jnp.int32),
        pltpu.VMEM((16, CW), jnp.int32),
        pltpu.VMEM((16, CW), jnp.int32),
        pltpu.VMEM((CW, D), jnp.float32),
        pltpu.VMEM((CW, D), jnp.float32),
        pltpu.VMEM_SHARED((NP, D), jnp.float32),
        pltpu.SemaphoreType.DMA,
        pltpu.SemaphoreType.DMA,
    ],
)
def _agg_kernel(src_hbm, dst_hbm, u_hbm, out_hbm,
                src_v, dsl_a, dsl_b, rows_a, rows_b, acc_sh,
                gsem_a, gsem_b):
    c = lax.axis_index("c")
    s = lax.axis_index("s")
    pltpu.sync_copy(src_hbm.at[c, s], src_v)

    zero = jnp.zeros((16,), jnp.float32)

    def fz(i, _):
        for q in range(8):
            rows_a[i, pl.ds(q * 16, 16)] = zero
        return 0

    lax.fori_loop(0, CW, fz, 0)
    for t in range(RPT // CW):
        pltpu.sync_copy(rows_a, acc_sh.at[pl.ds(s * RPT + t * CW, CW)])
    plsc.subcore_barrier()

    pltpu.async_copy(u_hbm.at[src_v.at[0]], rows_a, gsem_a)
    pltpu.async_copy(u_hbm.at[src_v.at[1]], rows_b, gsem_b)

    dslabs = (dsl_a, dsl_b)
    for b in range(NB):
        dsl = dslabs[b % 2]
        pltpu.sync_copy(dst_hbm.at[c, s].at[pl.ds(b * 16, 16)], dsl)

        def inner(t, _, b=b, dsl=dsl):
            j = b * 16 + 2 * t
            r = 2 * t
            pltpu.make_async_copy(u_hbm.at[src_v.at[j]], rows_a, gsem_a).wait()
            pltpu.sync_copy(rows_a, acc_sh.at[dsl.at[r]], add=True)

            @pl.when(j + 2 < CH)
            def _():
                pltpu.async_copy(u_hbm.at[src_v.at[j + 2]], rows_a, gsem_a)

            pltpu.make_async_copy(u_hbm.at[src_v.at[j + 1]], rows_b,
                                  gsem_b).wait()
            pltpu.sync_copy(rows_b, acc_sh.at[dsl.at[r + 1]], add=True)

            @pl.when(j + 3 < CH)
            def _():
                pltpu.async_copy(u_hbm.at[src_v.at[j + 3]], rows_b, gsem_b)

            return 0

        lax.fori_loop(0, 8, inner, 0)

    plsc.subcore_barrier()
    pltpu.sync_copy(acc_sh.at[pl.ds(s * RPT, RPT)],
                    out_hbm.at[c].at[pl.ds(s * RPT, RPT)])


def _deg_col(degp_ref):
    return degp_ref[0, :, 0:1] + degp_ref[1, :, 0:1] + 1.0


def _scale_in_body(degp_ref, x_ref, u_ref, degc_ref):
    deg = _deg_col(degp_ref)
    degc_ref[...] = deg
    u_ref[...] = x_ref[...] * lax.rsqrt(deg)


def _mid_body(degc_ref, u_ref, p_ref, v_ref):
    v_ref[...] = (u_ref[...] + p_ref[0] + p_ref[1]) / degc_ref[...]


def _out_body(degc_ref, v_ref, q_ref, w_ref, b_ref, o_ref):
    h2 = (v_ref[...] + q_ref[0] + q_ref[1]) * lax.rsqrt(degc_ref[...])
    o_ref[...] = lax.dot_general(
        h2, w_ref[...], (((1,), (1,)), ((), ())),
        preferred_element_type=jnp.float32) + b_ref[...]


_degp_spec = pl.BlockSpec((NC, RB, D), lambda i: (0, i, 0))
_row_spec = pl.BlockSpec((RB, D), lambda i: (i, 0))
_pair_spec = pl.BlockSpec((NC, RB, D), lambda i: (0, i, 0))

_degc_spec = pl.BlockSpec((RB, 1), lambda i: (i, 0))

_tc_scale_in = pl.pallas_call(
    _scale_in_body,
    grid=(NP // RB,),
    in_specs=[_degp_spec, _row_spec],
    out_specs=[_row_spec, _degc_spec],
    out_shape=[jax.ShapeDtypeStruct((NP, D), jnp.float32),
               jax.ShapeDtypeStruct((NP, 1), jnp.float32)],
)

_tc_mid = pl.pallas_call(
    _mid_body,
    grid=(NP // RB,),
    in_specs=[_degc_spec, _row_spec, _pair_spec],
    out_specs=_row_spec,
    out_shape=jax.ShapeDtypeStruct((NP, D), jnp.float32),
)

RBO = 1000

_tc_out = pl.pallas_call(
    _out_body,
    grid=(N // RBO,),
    in_specs=[
        pl.BlockSpec((RBO, 1), lambda i: (i, 0)),
        pl.BlockSpec((RBO, D), lambda i: (i, 0)),
        pl.BlockSpec((NC, RBO, D), lambda i: (0, i, 0)),
        pl.BlockSpec((D, D), lambda i: (0, 0)),
        pl.BlockSpec((1, D), lambda i: (0, 0)),
    ],
    out_specs=pl.BlockSpec((RBO, D), lambda i: (i, 0)),
    out_shape=jax.ShapeDtypeStruct((N, D), jnp.float32),
)


@jax.jit
def kernel(x, edge_index, W, b):
    ppw = CH * CW - E // (NC * NS)
    pad = jnp.broadcast_to(
        N + jnp.arange(ppw, dtype=edge_index.dtype), (NC, NS, ppw))
    src3 = jnp.concatenate(
        [edge_index[0].reshape(NC, NS, E // (NC * NS)), pad],
        axis=2).reshape(NC, NS, CH, CW)
    dst3 = jnp.concatenate(
        [edge_index[1].reshape(NC, NS, E // (NC * NS)), pad],
        axis=2).reshape(NC, NS, CH, CW)
    degp = _deg_kernel(dst3)
    u, degc = _tc_scale_in(degp, x)
    p = _agg_kernel(src3, dst3, u)
    v = _tc_mid(degc, u, p)
    q = _agg_kernel(src3, dst3, v)
    return _tc_out(degc, v, q, W, b.reshape(1, D))
